# baseline (speedup 1.0000x reference)
import functools

import jax
import jax.numpy as jnp
from jax.experimental import pallas as pl
from jax.experimental.pallas import tpu as pltpu

_L = 2
_E = 8
_TOPK = (2, 1)


def _layer_kernel(k, x_ref, m_ref, g_ref, b_ref, wr_ref, br_ref,
                  w1_ref, b1_ref, w2_ref, b2_ref, o_ref, h_scr, gate_scr):
    e = pl.program_id(0)

    @pl.when(e == 0)
    def _prep():
        x = x_ref[...]
        mu = jnp.mean(x, axis=-1, keepdims=True)
        xc = x - mu
        var = jnp.mean(xc * xc, axis=-1, keepdims=True)
        h = xc / jnp.sqrt(var + 1e-5) * g_ref[...] + b_ref[...]
        hb = h.astype(jnp.bfloat16)
        logits = jax.lax.dot_general(
            hb, wr_ref[...].astype(jnp.bfloat16), (((1,), (0,)), ((), ())),
            preferred_element_type=jnp.float32) + br_ref[...]
        mx = jnp.max(logits, axis=-1, keepdims=True)
        ex = jnp.exp(logits - mx)
        probs = ex / jnp.sum(ex, axis=-1, keepdims=True)
        iota = jax.lax.broadcasted_iota(jnp.int32, probs.shape, 1)
        m1 = jnp.max(probs, axis=-1, keepdims=True)
        a1 = jnp.min(jnp.where(probs == m1, iota, _E), axis=-1, keepdims=True)
        oh1 = (iota == a1).astype(x.dtype)
        if k == 1:
            gates = oh1
        else:
            probs2 = jnp.where(iota == a1, -jnp.inf, probs)
            m2 = jnp.max(probs2, axis=-1, keepdims=True)
            a2 = jnp.min(jnp.where(probs2 == m2, iota, _E),
                         axis=-1, keepdims=True)
            oh2 = (iota == a2).astype(x.dtype)
            gates = (m1 * oh1 + m2 * oh2) / (m1 + m2)
        gate_scr[...] = gates * m_ref[...]
        h_scr[...] = hb
        o_ref[...] = x

    T = x_ref.shape[0]
    CH = 512

    def _ffn_chunk(i, carry):
        sl = pl.ds(i * CH, CH)
        h = h_scr[sl, :]
        a = jax.lax.dot_general(h, w1_ref[0], (((1,), (0,)), ((), ())),
                                preferred_element_type=jnp.float32) + b1_ref[0]
        a = a * (1.0 / (1.0 + jnp.exp(-a)))
        eo = jax.lax.dot_general(a.astype(jnp.bfloat16), w2_ref[0],
                                 (((1,), (0,)), ((), ())),
                                 preferred_element_type=jnp.float32) + b2_ref[0]
        gates = gate_scr[sl, :]
        giota = jax.lax.broadcasted_iota(jnp.int32, gates.shape, 1)
        gcol = jnp.sum(jnp.where(giota == e, gates, 0.0),
                       axis=-1, keepdims=True)
        gcol = gcol.astype(jnp.bfloat16).astype(jnp.float32)
        eo = eo.astype(jnp.bfloat16).astype(jnp.float32)
        o_ref[sl, :] += gcol * eo
        return carry

    jax.lax.fori_loop(0, T // CH, _ffn_chunk, 0)


def _moe_layer(x2d, mask_f, g, b, wr, brr, w1, b1, w2, b2, k):
    T, H = x2d.shape
    F = w1.shape[-1]
    kern = functools.partial(_layer_kernel, k)
    return pl.pallas_call(
        kern,
        grid=(_E,),
        in_specs=[
            pl.BlockSpec((T, H), lambda e: (0, 0)),
            pl.BlockSpec((T, 1), lambda e: (0, 0)),
            pl.BlockSpec((1, H), lambda e: (0, 0)),
            pl.BlockSpec((1, H), lambda e: (0, 0)),
            pl.BlockSpec((H, _E), lambda e: (0, 0)),
            pl.BlockSpec((1, _E), lambda e: (0, 0)),
            pl.BlockSpec((1, H, F), lambda e: (e, 0, 0)),
            pl.BlockSpec((1, 1, F), lambda e: (e, 0, 0)),
            pl.BlockSpec((1, F, H), lambda e: (e, 0, 0)),
            pl.BlockSpec((1, 1, H), lambda e: (e, 0, 0)),
        ],
        out_specs=pl.BlockSpec((T, H), lambda e: (0, 0)),
        out_shape=jax.ShapeDtypeStruct((T, H), jnp.float32),
        scratch_shapes=[
            pltpu.VMEM((T, H), jnp.bfloat16),
            pltpu.VMEM((T, _E), jnp.float32),
        ],
        compiler_params=pltpu.CompilerParams(
            dimension_semantics=("arbitrary",)),
    )(x2d, mask_f, g, b, wr, brr, w1, b1, w2, b2)


def _pallas_layer(x, token_mask, g, b, wr, brr, w1, b1, w2, b2, k):
    B, S, H = x.shape
    T = B * S
    F = w1.shape[-1]
    mask_f = token_mask.reshape(T, 1).astype(jnp.float32)
    y = _moe_layer(
        x.reshape(T, H), mask_f,
        g.reshape(1, H), b.reshape(1, H),
        wr, brr.reshape(1, _E),
        w1.astype(jnp.bfloat16), b1.reshape(_E, 1, F),
        w2.astype(jnp.bfloat16), b2.reshape(_E, 1, H), k)
    return y.reshape(B, S, H)


def kernel(hidden_states, token_mask, ln_g, ln_b, Wr, br, W1, b1, W2, b2):
    x = hidden_states
    for l in range(_L):
        x = _pallas_layer(x, token_mask, ln_g[l], ln_b[l], Wr[l], br[l],
                          W1[l], b1[l], W2[l], b2[l], _TOPK[l])
    return x
